# baseline (speedup 1.0000x reference)
import functools

import jax
import jax.numpy as jnp
from jax import lax
from jax.experimental import pallas as pl
from jax.experimental.pallas import tpu as pltpu
from jax.experimental.pallas import tpu_sc as plsc

N = 10000
NE = 320000
NC = 2
NS = 16
CH = 128
NEP = 327680
ACC = 10240
RPT = ACC // NS

_MESH = dict(core_axis_name="c", subcore_axis_name="s", num_cores=NC,
             num_subcores=NS)


def _sc_degree(dst2d):
    cpt = NEP // (NC * NS) // CH

    @functools.partial(
        pl.kernel,
        out_type=jax.ShapeDtypeStruct((NC * NS, ACC), jnp.float32),
        mesh=plsc.VectorSubcoreMesh(**_MESH),
        compiler_params=pltpu.CompilerParams(needs_layout_passes=False),
        scratch_types=[
            pltpu.VMEM((cpt, CH), jnp.int32),
            pltpu.VMEM((ACC,), jnp.float32),
        ],
    )
    def k(dst_hbm, out_hbm, idx_v, hist_v):
        c = lax.axis_index("c")
        s = lax.axis_index("s")
        w = c * NS + s
        pltpu.sync_copy(dst_hbm.at[pl.ds(w * cpt, cpt)], idx_v)
        zeros16 = jnp.zeros((16,), jnp.float32)
        ones16 = jnp.ones((16,), jnp.float32)

        def zbody(i, carry):
            hist_v[pl.ds(i * 16, 16)] = zeros16
            return carry

        lax.fori_loop(0, ACC // 16, zbody, 0)

        def body(j, carry):
            def inner(k2, carry2):
                v = idx_v[j, pl.ds(k2 * 16, 16)]
                plsc.addupdate_scatter(hist_v, [v], ones16)
                return carry2

            lax.fori_loop(0, CH // 16, inner, 0)
            return carry

        lax.fori_loop(0, cpt, body, 0)
        pltpu.sync_copy(hist_v, out_hbm.at[w])

    return k(dst2d)


G = 40


def _agg_body(tbl_hbm, src_at, dst_at, zeros_hbm, out_hbm,
              idxs_v, idxd_v, rowsA, rowsB, acc_sh,
              gA, gB, sA, sB, osem, s, c, cpt):
    def _gather(jg, buf, sem):
        pltpu.async_copy(tbl_hbm.at[idxs_v.at[jg]], buf, sem)

    def _gather_wait(buf, sem):
        pltpu.make_async_copy(tbl_hbm.at[idxs_v.at[0]], buf, sem).wait()

    def _scat(jg, buf, sem):
        pltpu.async_copy(buf, acc_sh.at[idxd_v.at[jg]], sem, add=True)

    def _scat_wait(buf, sem):
        pltpu.make_async_copy(buf, acc_sh.at[idxd_v.at[0]], sem).wait()

    pltpu.async_copy(zeros_hbm, acc_sh.at[pl.ds(s * RPT, RPT)], osem).wait()
    plsc.subcore_barrier()

    def group(g, carry):
        pltpu.sync_copy(src_at(g), idxs_v)
        pltpu.sync_copy(dst_at(g), idxd_v)
        _gather(0, rowsA, gA)

        def pair(t, carry2):
            j0 = 2 * t
            _gather_wait(rowsA, gA)

            @pl.when(t > 0)
            def _():
                _scat_wait(rowsB, sB)

            _gather(j0 + 1, rowsB, gB)
            _scat(j0, rowsA, sA)
            _gather_wait(rowsB, gB)
            _scat_wait(rowsA, sA)

            @pl.when(t < G // 2 - 1)
            def _():
                _gather(j0 + 2, rowsA, gA)

            _scat(j0 + 1, rowsB, sB)
            return carry2

        lax.fori_loop(0, G // 2, pair, 0)
        _scat_wait(rowsB, sB)
        return carry

    lax.fori_loop(0, cpt // G, group, 0)
    plsc.subcore_barrier()
    pltpu.async_copy(acc_sh.at[pl.ds(s * RPT, RPT)],
                     out_hbm.at[c, pl.ds(s * RPT, RPT)], osem).wait()


def _sc_agg_edges(tbl, src2d, dst2d, zeros_b):
    cpt = NEP // (NC * NS) // CH

    @functools.partial(
        pl.kernel,
        out_type=jax.ShapeDtypeStruct((NC, ACC, 128), jnp.float32),
        mesh=plsc.VectorSubcoreMesh(**_MESH),
        scratch_types=[
            pltpu.VMEM((G, CH), jnp.int32),
            pltpu.VMEM((G, CH), jnp.int32),
            pltpu.VMEM((CH, 128), jnp.float32),
            pltpu.VMEM((CH, 128), jnp.float32),
            pltpu.VMEM_SHARED((ACC, 128), jnp.float32),
            pltpu.SemaphoreType.DMA,
            pltpu.SemaphoreType.DMA,
            pltpu.SemaphoreType.DMA,
            pltpu.SemaphoreType.DMA,
            pltpu.SemaphoreType.DMA,
        ],
    )
    def k(tbl_hbm, src_hbm, dst_hbm, zeros_hbm, out_hbm,
          idxs_v, idxd_v, rowsA, rowsB, acc_sh, gA, gB, sA, sB, osem):
        c = lax.axis_index("c")
        s = lax.axis_index("s")
        w = c * NS + s
        _agg_body(tbl_hbm,
                  lambda g: src_hbm.at[pl.ds((w * cpt) + g * G, G)],
                  lambda g: dst_hbm.at[pl.ds((w * cpt) + g * G, G)],
                  zeros_hbm, out_hbm, idxs_v, idxd_v, rowsA, rowsB,
                  acc_sh, gA, gB, sA, sB, osem, s, c, cpt)

    return k(tbl, src2d, dst2d, zeros_b)


def _sc_agg_feats(tbl2, src2, dst2d, zeros_b):
    cpt = NEP // NS // CH

    @functools.partial(
        pl.kernel,
        out_type=jax.ShapeDtypeStruct((NC, ACC, 128), jnp.float32),
        mesh=plsc.VectorSubcoreMesh(**_MESH),
        scratch_types=[
            pltpu.VMEM((G, CH), jnp.int32),
            pltpu.VMEM((G, CH), jnp.int32),
            pltpu.VMEM((CH, 128), jnp.float32),
            pltpu.VMEM((CH, 128), jnp.float32),
            pltpu.VMEM_SHARED((ACC, 128), jnp.float32),
            pltpu.SemaphoreType.DMA,
            pltpu.SemaphoreType.DMA,
            pltpu.SemaphoreType.DMA,
            pltpu.SemaphoreType.DMA,
            pltpu.SemaphoreType.DMA,
        ],
    )
    def k(tbl_hbm, src_hbm, dst_hbm, zeros_hbm, out_hbm,
          idxs_v, idxd_v, rowsA, rowsB, acc_sh, gA, gB, sA, sB, osem):
        c = lax.axis_index("c")
        s = lax.axis_index("s")
        _agg_body(tbl_hbm,
                  lambda g: src_hbm.at[c, pl.ds(s * cpt + g * G, G)],
                  lambda g: dst_hbm.at[pl.ds(s * cpt + g * G, G)],
                  zeros_hbm, out_hbm, idxs_v, idxd_v, rowsA, rowsB,
                  acc_sh, gA, gB, sA, sB, osem, s, c, cpt)

    return k(tbl2, src2, dst2d, zeros_b)


BR = 10000


def _tc_prep(hist_t, x):
    def body(hp, x_ref, y1_ref, dinv_ref):
        deg = jnp.sum(hp[...], axis=1, keepdims=True) + 1.0
        dinv = lax.rsqrt(deg)
        y1_ref[...] = x_ref[...] * dinv
        dinv_ref[...] = jnp.broadcast_to(dinv, (BR, 128))

    return pl.pallas_call(
        body,
        grid=(N // BR,),
        in_specs=[
            pl.BlockSpec((BR, NC * NS), lambda i: (i, 0)),
            pl.BlockSpec((BR, 128), lambda i: (i, 0)),
        ],
        out_specs=[
            pl.BlockSpec((BR, 128), lambda i: (i, 0)),
            pl.BlockSpec((BR, 128), lambda i: (i, 0)),
        ],
        out_shape=[jax.ShapeDtypeStruct((N, 128), jnp.float32)] * 2,
    )(hist_t, x)


def _tc_layer1(s1, x, dinvb, W1, b1):
    def body(sp0, sp1, x_ref, dv_ref, w_ref, b_ref, h_ref, y2_ref):
        dv = dv_ref[...]
        ax = dv * (sp0[0] + sp1[0]) + dv * dv * x_ref[...]
        h = jnp.dot(ax, w_ref[...], preferred_element_type=jnp.float32) + b_ref[...]
        h_ref[...] = h
        y2_ref[...] = jnp.concatenate([dv, dv], axis=1) * h

    return pl.pallas_call(
        body,
        grid=(N // BR,),
        in_specs=[
            pl.BlockSpec((1, BR, 128), lambda i: (0, i, 0)),
            pl.BlockSpec((1, BR, 128), lambda i: (1, i, 0)),
            pl.BlockSpec((BR, 128), lambda i: (i, 0)),
            pl.BlockSpec((BR, 128), lambda i: (i, 0)),
            pl.BlockSpec((128, 256), lambda i: (0, 0)),
            pl.BlockSpec((1, 256), lambda i: (0, 0)),
        ],
        out_specs=[
            pl.BlockSpec((BR, 256), lambda i: (i, 0)),
            pl.BlockSpec((BR, 256), lambda i: (i, 0)),
        ],
        out_shape=[jax.ShapeDtypeStruct((N, 256), jnp.float32)] * 2,
    )(s1, s1, x, dinvb, W1, b1)


def _tc_layer23(s2, h, dinvb, W_mean, b_mean, W_logstd, b_logstd):
    def body(sa, sb, h_ref, dv_ref, wm_ref, bm_ref, wl_ref, bl_ref,
             mean_ref, logstd_ref):
        dv = dv_ref[...]
        dv2 = jnp.concatenate([dv, dv], axis=1)
        ah = dv2 * jnp.concatenate([sa[0], sb[0]], axis=1) + dv2 * dv2 * h_ref[...]
        mean_ref[...] = (
            jnp.dot(ah, wm_ref[...], preferred_element_type=jnp.float32) + bm_ref[...]
        )
        logstd_ref[...] = (
            jnp.dot(ah, wl_ref[...], preferred_element_type=jnp.float32) + bl_ref[...]
        )

    return pl.pallas_call(
        body,
        grid=(N // BR,),
        in_specs=[
            pl.BlockSpec((1, BR, 128), lambda i: (0, i, 0)),
            pl.BlockSpec((1, BR, 128), lambda i: (1, i, 0)),
            pl.BlockSpec((BR, 256), lambda i: (i, 0)),
            pl.BlockSpec((BR, 128), lambda i: (i, 0)),
            pl.BlockSpec((256, 128), lambda i: (0, 0)),
            pl.BlockSpec((1, 128), lambda i: (0, 0)),
            pl.BlockSpec((256, 128), lambda i: (0, 0)),
            pl.BlockSpec((1, 128), lambda i: (0, 0)),
        ],
        out_specs=[
            pl.BlockSpec((BR, 128), lambda i: (i, 0)),
            pl.BlockSpec((BR, 128), lambda i: (i, 0)),
        ],
        out_shape=[jax.ShapeDtypeStruct((N, 128), jnp.float32)] * 2,
    )(s2, s2, h, dinvb, W_mean, b_mean, W_logstd, b_logstd)


def kernel(x, edge_index, W1, b1, W_mean, b_mean, W_logstd, b_logstd):
    x = x.astype(jnp.float32)
    src = edge_index[0].astype(jnp.int32)
    dst = edge_index[1].astype(jnp.int32)
    pad = NEP - NE
    srcp = jnp.concatenate([src, jnp.arange(pad, dtype=jnp.int32) % N])
    dump = N + jnp.arange(pad, dtype=jnp.int32) % (ACC - N)
    dstp = jnp.concatenate([dst, dump])
    src2d = srcp.reshape(NEP // CH, CH)
    dst2d = dstp.reshape(NEP // CH, CH)
    src2 = jnp.stack([srcp * 2, srcp * 2 + 1]).reshape(NC, NEP // CH, CH)
    zeros_b = jnp.zeros((RPT, 128), jnp.float32)

    hist = _sc_degree(dst2d)
    y1, dinvb = _tc_prep(hist.T, x)
    s1 = _sc_agg_edges(y1, src2d, dst2d, zeros_b)
    h, y2 = _tc_layer1(s1, x, dinvb, W1,
                       b1.reshape(1, 256))
    s2 = _sc_agg_feats(y2.reshape(2 * N, 128), src2, dst2d,
                       zeros_b)
    mean, logstd = _tc_layer23(s2, h, dinvb,
                               W_mean, b_mean.reshape(1, 128),
                               W_logstd, b_logstd.reshape(1, 128))
    return (mean, logstd)


# speedup vs baseline: 1.0127x; 1.0127x over previous
import functools

import jax
import jax.numpy as jnp
from jax import lax
from jax.experimental import pallas as pl
from jax.experimental.pallas import tpu as pltpu
from jax.experimental.pallas import tpu_sc as plsc

N = 10000
NE = 320000
NC = 2
NS = 16
CH = 128
NEP = 327680
ACC = 10240
RPT = ACC // NS

_MESH = dict(core_axis_name="c", subcore_axis_name="s", num_cores=NC,
             num_subcores=NS)


def _sc_degree(dst2d):
    cpt = NEP // (NC * NS) // CH

    @functools.partial(
        pl.kernel,
        out_type=jax.ShapeDtypeStruct((NC * NS, ACC), jnp.float32),
        mesh=plsc.VectorSubcoreMesh(**_MESH),
        compiler_params=pltpu.CompilerParams(needs_layout_passes=False),
        scratch_types=[
            pltpu.VMEM((cpt, CH), jnp.int32),
            pltpu.VMEM((ACC,), jnp.float32),
        ],
    )
    def k(dst_hbm, out_hbm, idx_v, hist_v):
        c = lax.axis_index("c")
        s = lax.axis_index("s")
        w = c * NS + s
        pltpu.sync_copy(dst_hbm.at[pl.ds(w * cpt, cpt)], idx_v)
        zeros16 = jnp.zeros((16,), jnp.float32)
        ones16 = jnp.ones((16,), jnp.float32)

        def zbody(i, carry):
            hist_v[pl.ds(i * 16, 16)] = zeros16
            return carry

        lax.fori_loop(0, ACC // 16, zbody, 0)

        def body(j, carry):
            def inner(k2, carry2):
                v = idx_v[j, pl.ds(k2 * 16, 16)]
                plsc.addupdate_scatter(hist_v, [v], ones16)
                return carry2

            lax.fori_loop(0, CH // 16, inner, 0)
            return carry

        lax.fori_loop(0, cpt, body, 0)
        pltpu.sync_copy(hist_v, out_hbm.at[w])

    return k(dst2d)


G = 40


def _agg_body(tbl_hbm, src_at, dst_at, zeros_hbm, out_hbm,
              idxs_v, idxd_v, rowsA, rowsB, acc_sh,
              gA, gB, sA, sB, osem, s, c, cpt):
    def _gather(jg, buf, sem):
        pltpu.async_copy(tbl_hbm.at[idxs_v.at[jg]], buf, sem)

    def _gather_wait(buf, sem):
        pltpu.make_async_copy(tbl_hbm.at[idxs_v.at[0]], buf, sem).wait()

    def _scat(jg, buf, sem):
        pltpu.async_copy(buf, acc_sh.at[idxd_v.at[jg]], sem, add=True)

    def _scat_wait(buf, sem):
        pltpu.make_async_copy(buf, acc_sh.at[idxd_v.at[0]], sem).wait()

    pltpu.async_copy(zeros_hbm, acc_sh.at[pl.ds(s * RPT, RPT)], osem).wait()
    plsc.subcore_barrier()

    def group(g, carry):
        pltpu.sync_copy(src_at(g), idxs_v)
        pltpu.sync_copy(dst_at(g), idxd_v)
        _gather(0, rowsA, gA)

        def pair(t, carry2):
            j0 = 2 * t
            _gather_wait(rowsA, gA)

            @pl.when(t > 0)
            def _():
                _scat_wait(rowsB, sB)

            _gather(j0 + 1, rowsB, gB)
            _scat(j0, rowsA, sA)
            _gather_wait(rowsB, gB)
            _scat_wait(rowsA, sA)

            @pl.when(t < G // 2 - 1)
            def _():
                _gather(j0 + 2, rowsA, gA)

            _scat(j0 + 1, rowsB, sB)
            return carry2

        lax.fori_loop(0, G // 2, pair, 0)
        _scat_wait(rowsB, sB)
        return carry

    lax.fori_loop(0, cpt // G, group, 0)
    plsc.subcore_barrier()
    pltpu.async_copy(acc_sh.at[pl.ds(s * RPT, RPT)],
                     out_hbm.at[c, pl.ds(s * RPT, RPT)], osem).wait()


def _sc_agg_edges(tbl, src2d, dst2d, zeros_b):
    cpt = NEP // (NC * NS) // CH

    @functools.partial(
        pl.kernel,
        out_type=jax.ShapeDtypeStruct((NC, ACC, 128), jnp.float32),
        mesh=plsc.VectorSubcoreMesh(**_MESH),
        scratch_types=[
            pltpu.VMEM((G, CH), jnp.int32),
            pltpu.VMEM((G, CH), jnp.int32),
            pltpu.VMEM((CH, 128), jnp.float32),
            pltpu.VMEM((CH, 128), jnp.float32),
            pltpu.VMEM_SHARED((ACC, 128), jnp.float32),
            pltpu.SemaphoreType.DMA,
            pltpu.SemaphoreType.DMA,
            pltpu.SemaphoreType.DMA,
            pltpu.SemaphoreType.DMA,
            pltpu.SemaphoreType.DMA,
        ],
    )
    def k(tbl_hbm, src_hbm, dst_hbm, zeros_hbm, out_hbm,
          idxs_v, idxd_v, rowsA, rowsB, acc_sh, gA, gB, sA, sB, osem):
        c = lax.axis_index("c")
        s = lax.axis_index("s")
        w = c * NS + s
        _agg_body(tbl_hbm,
                  lambda g: src_hbm.at[pl.ds((w * cpt) + g * G, G)],
                  lambda g: dst_hbm.at[pl.ds((w * cpt) + g * G, G)],
                  zeros_hbm, out_hbm, idxs_v, idxd_v, rowsA, rowsB,
                  acc_sh, gA, gB, sA, sB, osem, s, c, cpt)

    return k(tbl, src2d, dst2d, zeros_b)


def _sc_agg_feats(tbl2, src2, dst2d, zeros_b):
    cpt = NEP // NS // CH

    @functools.partial(
        pl.kernel,
        out_type=jax.ShapeDtypeStruct((NC, ACC, 128), jnp.float32),
        mesh=plsc.VectorSubcoreMesh(**_MESH),
        scratch_types=[
            pltpu.VMEM((G, CH), jnp.int32),
            pltpu.VMEM((G, CH), jnp.int32),
            pltpu.VMEM((CH, 128), jnp.float32),
            pltpu.VMEM((CH, 128), jnp.float32),
            pltpu.VMEM_SHARED((ACC, 128), jnp.float32),
            pltpu.SemaphoreType.DMA,
            pltpu.SemaphoreType.DMA,
            pltpu.SemaphoreType.DMA,
            pltpu.SemaphoreType.DMA,
            pltpu.SemaphoreType.DMA,
        ],
    )
    def k(tbl_hbm, src_hbm, dst_hbm, zeros_hbm, out_hbm,
          idxs_v, idxd_v, rowsA, rowsB, acc_sh, gA, gB, sA, sB, osem):
        c = lax.axis_index("c")
        s = lax.axis_index("s")
        _agg_body(tbl_hbm,
                  lambda g: src_hbm.at[c, pl.ds(s * cpt + g * G, G)],
                  lambda g: dst_hbm.at[pl.ds(s * cpt + g * G, G)],
                  zeros_hbm, out_hbm, idxs_v, idxd_v, rowsA, rowsB,
                  acc_sh, gA, gB, sA, sB, osem, s, c, cpt)

    return k(tbl2, src2, dst2d, zeros_b)


BR = 5000


def _tc_prep(hist_t, x):
    def body(hp, x_ref, y1_ref, dinv_ref):
        deg = jnp.sum(hp[...], axis=1, keepdims=True) + 1.0
        dinv = lax.rsqrt(deg)
        y1_ref[...] = x_ref[...] * dinv
        dinv_ref[...] = jnp.broadcast_to(dinv, (BR, 128))

    return pl.pallas_call(
        body,
        grid=(N // BR,),
        in_specs=[
            pl.BlockSpec((BR, NC * NS), lambda i: (i, 0)),
            pl.BlockSpec((BR, 128), lambda i: (i, 0)),
        ],
        out_specs=[
            pl.BlockSpec((BR, 128), lambda i: (i, 0)),
            pl.BlockSpec((BR, 128), lambda i: (i, 0)),
        ],
        out_shape=[jax.ShapeDtypeStruct((N, 128), jnp.float32)] * 2,
    )(hist_t, x)


def _tc_layer1(s1, x, dinvb, W1, b1):
    def body(sp0, sp1, x_ref, dv_ref, w_ref, b_ref, h_ref, y2_ref):
        dv = dv_ref[...]
        ax = dv * (sp0[0] + sp1[0]) + dv * dv * x_ref[...]
        h = jnp.dot(ax, w_ref[...], preferred_element_type=jnp.float32) + b_ref[...]
        h_ref[...] = h
        y2_ref[...] = jnp.concatenate([dv, dv], axis=1) * h

    return pl.pallas_call(
        body,
        grid=(N // BR,),
        in_specs=[
            pl.BlockSpec((1, BR, 128), lambda i: (0, i, 0)),
            pl.BlockSpec((1, BR, 128), lambda i: (1, i, 0)),
            pl.BlockSpec((BR, 128), lambda i: (i, 0)),
            pl.BlockSpec((BR, 128), lambda i: (i, 0)),
            pl.BlockSpec((128, 256), lambda i: (0, 0)),
            pl.BlockSpec((1, 256), lambda i: (0, 0)),
        ],
        out_specs=[
            pl.BlockSpec((BR, 256), lambda i: (i, 0)),
            pl.BlockSpec((BR, 256), lambda i: (i, 0)),
        ],
        out_shape=[jax.ShapeDtypeStruct((N, 256), jnp.float32)] * 2,
    )(s1, s1, x, dinvb, W1, b1)


def _tc_layer23(s2, h, dinvb, W_mean, b_mean, W_logstd, b_logstd):
    def body(sa, sb, h_ref, dv_ref, wm_ref, bm_ref, wl_ref, bl_ref,
             mean_ref, logstd_ref):
        dv = dv_ref[...]
        dv2 = jnp.concatenate([dv, dv], axis=1)
        ah = dv2 * jnp.concatenate([sa[0], sb[0]], axis=1) + dv2 * dv2 * h_ref[...]
        mean_ref[...] = (
            jnp.dot(ah, wm_ref[...], preferred_element_type=jnp.float32) + bm_ref[...]
        )
        logstd_ref[...] = (
            jnp.dot(ah, wl_ref[...], preferred_element_type=jnp.float32) + bl_ref[...]
        )

    return pl.pallas_call(
        body,
        grid=(N // BR,),
        in_specs=[
            pl.BlockSpec((1, BR, 128), lambda i: (0, i, 0)),
            pl.BlockSpec((1, BR, 128), lambda i: (1, i, 0)),
            pl.BlockSpec((BR, 256), lambda i: (i, 0)),
            pl.BlockSpec((BR, 128), lambda i: (i, 0)),
            pl.BlockSpec((256, 128), lambda i: (0, 0)),
            pl.BlockSpec((1, 128), lambda i: (0, 0)),
            pl.BlockSpec((256, 128), lambda i: (0, 0)),
            pl.BlockSpec((1, 128), lambda i: (0, 0)),
        ],
        out_specs=[
            pl.BlockSpec((BR, 128), lambda i: (i, 0)),
            pl.BlockSpec((BR, 128), lambda i: (i, 0)),
        ],
        out_shape=[jax.ShapeDtypeStruct((N, 128), jnp.float32)] * 2,
    )(s2, s2, h, dinvb, W_mean, b_mean, W_logstd, b_logstd)


def kernel(x, edge_index, W1, b1, W_mean, b_mean, W_logstd, b_logstd):
    x = x.astype(jnp.float32)
    src = edge_index[0].astype(jnp.int32)
    dst = edge_index[1].astype(jnp.int32)
    pad = NEP - NE
    srcp = jnp.concatenate([src, jnp.arange(pad, dtype=jnp.int32) % N])
    dump = N + jnp.arange(pad, dtype=jnp.int32) % (ACC - N)
    dstp = jnp.concatenate([dst, dump])
    src2d = srcp.reshape(NEP // CH, CH)
    dst2d = dstp.reshape(NEP // CH, CH)
    src2 = jnp.stack([srcp * 2, srcp * 2 + 1]).reshape(NC, NEP // CH, CH)
    zeros_b = jnp.zeros((RPT, 128), jnp.float32)

    hist = _sc_degree(dst2d)
    y1, dinvb = _tc_prep(hist.T, x)
    s1 = _sc_agg_edges(y1, src2d, dst2d, zeros_b)
    h, y2 = _tc_layer1(s1, x, dinvb, W1,
                       b1.reshape(1, 256))
    s2 = _sc_agg_feats(y2.reshape(2 * N, 128), src2, dst2d,
                       zeros_b)
    mean, logstd = _tc_layer23(s2, h, dinvb,
                               W_mean, b_mean.reshape(1, 128),
                               W_logstd, b_logstd.reshape(1, 128))
    return (mean, logstd)
